# Initial kernel scaffold; baseline (speedup 1.0000x reference)
import jax
import jax.numpy as jnp
from jax.experimental import pallas as pl


def kernel(x):
    raise NotImplementedError("write your pallas kernel here")


# speedup vs baseline: 1.2398x; 1.2398x over previous
import functools

import jax
import jax.numpy as jnp
from jax import lax
from jax.experimental import pallas as pl
from jax.experimental.pallas import tpu as pltpu
from jax.experimental.pallas import tpu_sc as plsc

L = 16
NROWS = 8192
NCOLS = 4096
NWORK = 32
ROWS_PER_W = NROWS // NWORK
R = 8
NCHUNK = ROWS_PER_W // R
CVEC = NCOLS // L
BITS = 30

_mesh = plsc.VectorSubcoreMesh(core_axis_name="c", subcore_axis_name="s")

_GDN = lax.GatherDimensionNumbers(
    offset_dims=(), collapsed_slice_dims=(0,), start_index_map=(0,)
)


def _shuf(v, idx):
    return lax.gather(
        v,
        idx[:, None],
        dimension_numbers=_GDN,
        slice_sizes=(1,),
        mode=lax.GatherScatterMode.PROMISE_IN_BOUNDS,
    )


def _xlane(v, op):
    r = v
    for k in (1, 2, 4, 8):
        idx = lax.iota(jnp.int32, L) ^ k
        r = op(r, _shuf(r, idx))
    return r


@functools.partial(
    pl.kernel,
    mesh=_mesh,
    out_type=jax.ShapeDtypeStruct((NROWS, NCOLS), jnp.float32),
    scratch_types=[
        pltpu.VMEM((R, NCOLS), jnp.float32),
        pltpu.VMEM((R, NCOLS), jnp.float32),
    ],
)
def _sparsemax_sc(x_hbm, out_hbm, in_v, out_v):
    wid = lax.axis_index("s") * 2 + lax.axis_index("c")
    base_row = wid * ROWS_PER_W

    def chunk_body(g, _):
        row0 = base_row + g * R
        pltpu.sync_copy(x_hbm.at[pl.ds(row0, R)], in_v)

        def row_body(r, _):
            row = in_v.at[r]

            def max_body(i, m):
                return jnp.maximum(m, row[pl.ds(i * L, L)])

            m = lax.fori_loop(
                0, CVEC, max_body, jnp.full((L,), -jnp.inf, jnp.float32)
            )
            top = _xlane(m, jnp.maximum)

            def bis_body(b, carry):
                lo, hi = carry
                mid = 0.5 * (lo + hi)

                def sum_body(i, acc):
                    v = row[pl.ds(i * L, L)]
                    return acc + jnp.maximum(v - mid, 0.0)

                s = _xlane(
                    lax.fori_loop(0, CVEC, sum_body, jnp.zeros((L,), jnp.float32)),
                    jnp.add,
                )
                big = s > 1.0
                return jnp.where(big, mid, lo), jnp.where(big, hi, mid)

            lo, hi = lax.fori_loop(0, BITS, bis_body, (top - 1.0, top))
            tau = 0.5 * (lo + hi)

            orow = out_v.at[r]

            def out_body(i, _):
                v = row[pl.ds(i * L, L)]
                orow[pl.ds(i * L, L)] = jnp.maximum(v - tau, 0.0)
                return 0

            lax.fori_loop(0, CVEC, out_body, 0)
            return 0

        lax.fori_loop(0, R, row_body, 0)
        pltpu.sync_copy(out_v, out_hbm.at[pl.ds(row0, R)])
        return 0

    lax.fori_loop(0, NCHUNK, chunk_body, 0)


def kernel(x):
    out = _sparsemax_sc(x.reshape(NROWS, NCOLS))
    return out.reshape(x.shape)


# speedup vs baseline: 7.1397x; 5.7586x over previous
import functools

import jax
import jax.numpy as jnp
from jax import lax
from jax.experimental import pallas as pl
from jax.experimental.pallas import tpu as pltpu
from jax.experimental.pallas import tpu_sc as plsc

L = 16
NROWS = 8192
NCOLS = 4096
NWORK = 32
ROWS_PER_W = NROWS // NWORK
R = 8
NCHUNK = ROWS_PER_W // R
CVEC = NCOLS // L
U = 8
NITER = CVEC // U
BITS = 22

_mesh = plsc.VectorSubcoreMesh(core_axis_name="c", subcore_axis_name="s")

_GDN = lax.GatherDimensionNumbers(
    offset_dims=(), collapsed_slice_dims=(0,), start_index_map=(0,)
)


def _shuf(v, idx):
    return lax.gather(
        v,
        idx[:, None],
        dimension_numbers=_GDN,
        slice_sizes=(1,),
        mode=lax.GatherScatterMode.PROMISE_IN_BOUNDS,
    )


def _xlane(v, op):
    r = v
    for k in (1, 2, 4, 8):
        idx = lax.iota(jnp.int32, L) ^ k
        r = op(r, _shuf(r, idx))
    return r


@functools.partial(
    pl.kernel,
    mesh=_mesh,
    out_type=jax.ShapeDtypeStruct((NROWS, NCOLS), jnp.float32),
    scratch_types=[
        pltpu.VMEM((R, NCOLS), jnp.float32),
        pltpu.VMEM((R, NCOLS), jnp.float32),
    ],
)
def _sparsemax_sc(x_hbm, out_hbm, in_v, out_v):
    wid = lax.axis_index("s") * 2 + lax.axis_index("c")
    base_row = wid * ROWS_PER_W

    def chunk_body(g, _):
        row0 = base_row + g * R
        pltpu.sync_copy(x_hbm.at[pl.ds(row0, R)], in_v)

        def row_body(r, _):
            row = in_v.at[r]

            def max_body(i, ms):
                return tuple(
                    jnp.maximum(ms[j], row[pl.ds(i * (U * L) + j * L, L)])
                    for j in range(U)
                )

            ms = lax.fori_loop(
                0, NITER, max_body,
                tuple(jnp.full((L,), -jnp.inf, jnp.float32) for _ in range(U)),
            )
            m = ms[0]
            for j in range(1, U):
                m = jnp.maximum(m, ms[j])
            top = _xlane(m, jnp.maximum)

            def bis_body(b, carry):
                lo, hi = carry
                mid = 0.5 * (lo + hi)

                def sum_body(i, accs):
                    return tuple(
                        accs[j]
                        + jnp.maximum(row[pl.ds(i * (U * L) + j * L, L)] - mid, 0.0)
                        for j in range(U)
                    )

                accs = lax.fori_loop(
                    0, NITER, sum_body,
                    tuple(jnp.zeros((L,), jnp.float32) for _ in range(U)),
                )
                a = accs[0]
                for j in range(1, U):
                    a = a + accs[j]
                s = _xlane(a, jnp.add)
                big = s > 1.0
                return jnp.where(big, mid, lo), jnp.where(big, hi, mid)

            lo, hi = lax.fori_loop(0, BITS, bis_body, (top - 1.0, top))
            tau = 0.5 * (lo + hi)

            orow = out_v.at[r]

            def out_body(i, _):
                for j in range(U):
                    off = i * (U * L) + j * L
                    orow[pl.ds(off, L)] = jnp.maximum(row[pl.ds(off, L)] - tau, 0.0)
                return 0

            lax.fori_loop(0, NITER, out_body, 0)
            return 0

        lax.fori_loop(0, R, row_body, 0)
        pltpu.sync_copy(out_v, out_hbm.at[pl.ds(row0, R)])
        return 0

    lax.fori_loop(0, NCHUNK, chunk_body, 0)


def kernel(x):
    out = _sparsemax_sc(x.reshape(NROWS, NCOLS))
    return out.reshape(x.shape)
